# baseline (speedup 1.0000x reference)
import jax
import jax.numpy as jnp
from jax.experimental import pallas as pl
from jax.experimental.pallas import tpu as pltpu

_R, _C = 128, 8192
_BLK = 16


def _body(y_ref, t_ref, out_ref):
    i = pl.program_id(0)
    y = y_ref[...]
    t = t_ref[...]
    a = jnp.where(t == 1.0, y, 1.0 - y)
    loss = (a - 1.0) * jnp.log(a)
    s = jnp.sum(loss)

    @pl.when(i == 0)
    def _init():
        out_ref[0, 0] = 0.0

    out_ref[0, 0] += s


def kernel(y, target):
    out = pl.pallas_call(
        _body,
        grid=(_R // _BLK,),
        in_specs=[
            pl.BlockSpec((_BLK, _C), lambda i: (i, 0)),
            pl.BlockSpec((_BLK, _C), lambda i: (i, 0)),
        ],
        out_specs=pl.BlockSpec((1, 1), lambda i: (0, 0), memory_space=pltpu.SMEM),
        out_shape=jax.ShapeDtypeStruct((1, 1), jnp.float32),
    )(y, target)
    return out[0, 0] / (_R * _C)
